# Initial kernel scaffold; baseline (speedup 1.0000x reference)
import functools

import jax
import jax.numpy as jnp
from jax.experimental import pallas as pl
from jax.experimental.pallas import tpu as pltpu
from jax.experimental.pallas import tpu_sc as plsc

N = 10000
E = 320000
F = 128
R = 200
B = 8
S = 16
T = 128
FW = F + 16
NC = 2
NS = 16
NW = NC * NS
C = 128
P = 348160
NT = P // T
NPAD = 10240

_mesh = plsc.VectorSubcoreMesh(core_axis_name="c", subcore_axis_name="s")


@functools.partial(
    pl.kernel,
    out_type=jax.ShapeDtypeStruct((P, F), jnp.float32),
    mesh=_mesh,
    scratch_types=[
        pltpu.VMEM((C,), jnp.int32),
        pltpu.VMEM((C, F), jnp.float32),
        pltpu.SemaphoreType.DMA,
    ],
)
def _sc_gather(x_hbm, idx_hbm, out_hbm, idx_v, rows_v, sem):
    wid = jax.lax.axis_index("s") * NC + jax.lax.axis_index("c")
    per_w = P // NW
    base0 = wid * per_w

    def body(i, carry):
        base = base0 + i * C
        pltpu.sync_copy(idx_hbm.at[pl.ds(base, C)], idx_v)
        pltpu.async_copy(x_hbm.at[idx_v], rows_v, sem).wait()
        pltpu.sync_copy(rows_v, out_hbm.at[pl.ds(base, C)])
        return carry

    jax.lax.fori_loop(0, per_w // C, body, 0)


@functools.partial(
    pl.kernel,
    out_type=jax.ShapeDtypeStruct((NC, NPAD, FW), jnp.float32),
    mesh=_mesh,
    scratch_types=[
        pltpu.VMEM((C,), jnp.int32),
        pltpu.VMEM((C, FW), jnp.float32),
        pltpu.VMEM_SHARED((NPAD, FW), jnp.float32),
    ],
)
def _sc_scatter(msg_hbm, dst_hbm, zeros_hbm, out_hbm, idx_v, rows_v, acc_sh):
    c = jax.lax.axis_index("c")
    s = jax.lax.axis_index("s")
    wid = s * NC + c
    rows_per_tile = NPAD // NS
    r0 = s * rows_per_tile
    pltpu.sync_copy(
        zeros_hbm.at[pl.ds(r0, rows_per_tile)],
        acc_sh.at[pl.ds(r0, rows_per_tile)],
    )
    plsc.subcore_barrier()
    per_w = P // NW
    base0 = wid * per_w

    def body(i, carry):
        base = base0 + i * C
        pltpu.sync_copy(dst_hbm.at[pl.ds(base, C)], idx_v)
        pltpu.sync_copy(msg_hbm.at[pl.ds(base, C)], rows_v)
        pltpu.sync_copy(rows_v, acc_sh.at[idx_v], add=True)
        return carry

    jax.lax.fori_loop(0, per_w // C, body, 0)
    plsc.subcore_barrier()
    pltpu.sync_copy(
        acc_sh.at[pl.ds(r0, rows_per_tile)],
        out_hbm.at[c].at[pl.ds(r0, rows_per_tile)],
    )


def _tc_bmm(rel_ref, xs_ref, bd_ref, out_ref):
    m = jax.lax.dot_general(
        xs_ref[...],
        bd_ref[0],
        (((1,), (0,)), ((), ())),
        preferred_element_type=jnp.float32,
        precision=jax.lax.Precision.HIGHEST,
    )
    out_ref[:, :F] = m
    tail = jax.lax.broadcasted_iota(jnp.int32, (T, FW - F), 1)
    out_ref[:, F:] = (tail == 0).astype(jnp.float32)


def _tc_combine(a_ref, b_ref, o_ref):
    acc = a_ref[...] + b_ref[...]
    deg = jnp.maximum(acc[:, F:F + 1], 1.0)
    o_ref[...] = acc[:, :F] / deg


def kernel(x, edge_index, etype, weight):
    src = edge_index[0]
    dst = edge_index[1]

    et_s, src_s, dst_s = jax.lax.sort([etype, src, dst], num_keys=1)
    off = jnp.searchsorted(et_s, jnp.arange(R + 1, dtype=jnp.int32)).astype(
        jnp.int32
    )
    cnt = off[1:] - off[:-1]
    pcnt = ((cnt + T - 1) // T) * T
    poff = jnp.concatenate(
        [jnp.zeros((1,), jnp.int32), jnp.cumsum(pcnt).astype(jnp.int32)]
    )
    pad_before = poff[:-1] - off[:-1]
    bnd = jnp.zeros((E,), jnp.int32).at[off[:-1]].max(pad_before, mode="drop")
    pos = jnp.arange(E, dtype=jnp.int32) + jax.lax.cummax(bnd)

    i_p = jnp.arange(P, dtype=jnp.int32)
    src_p = ((i_p * 37) % N).at[pos].set(src_s, unique_indices=True)
    dst_p = (N + (i_p % (NPAD - N))).at[pos].set(dst_s, unique_indices=True)

    tstart = jnp.arange(NT, dtype=jnp.int32) * T
    tile_rel = jnp.clip(
        jnp.searchsorted(poff, tstart, side="right").astype(jnp.int32) - 1,
        0,
        R - 1,
    )

    w4 = weight.reshape(R, B, S, S)
    eye = jnp.eye(B, dtype=weight.dtype)
    bd = (w4[:, :, :, None, :] * eye[None, :, None, :, None]).reshape(R, F, F)

    xs = _sc_gather(x, src_p)

    grid_spec = pltpu.PrefetchScalarGridSpec(
        num_scalar_prefetch=1,
        grid=(NT,),
        in_specs=[
            pl.BlockSpec((T, F), lambda t, rel: (t, 0)),
            pl.BlockSpec((1, F, F), lambda t, rel: (rel[t], 0, 0)),
        ],
        out_specs=pl.BlockSpec((T, FW), lambda t, rel: (t, 0)),
    )
    msg = pl.pallas_call(
        _tc_bmm,
        grid_spec=grid_spec,
        out_shape=jax.ShapeDtypeStruct((P, FW), jnp.float32),
    )(tile_rel, xs, bd)

    zeros = jnp.zeros((NPAD, FW), jnp.float32)
    parts = _sc_scatter(msg, dst_p, zeros)

    RT = 256
    h_pad = pl.pallas_call(
        _tc_combine,
        grid=(NPAD // RT,),
        in_specs=[
            pl.BlockSpec((RT, FW), lambda i: (i, 0)),
            pl.BlockSpec((RT, FW), lambda i: (i, 0)),
        ],
        out_specs=pl.BlockSpec((RT, F), lambda i: (i, 0)),
        out_shape=jax.ShapeDtypeStruct((NPAD, F), jnp.float32),
    )(parts[0], parts[1])
    return h_pad[:N]


# speedup vs baseline: 4.8258x; 4.8258x over previous
import functools

import jax
import jax.numpy as jnp
from jax.experimental import pallas as pl
from jax.experimental.pallas import tpu as pltpu
from jax.experimental.pallas import tpu_sc as plsc

N = 10000
E = 320000
F = 128
R = 200
B = 8
S = 16
T = 128
FW = F + 16
NC = 2
NS = 16
NW = NC * NS
C = 128
P = 348160
NT = P // T
NPAD = 10240

@functools.cache
def _sc_kernels():
    mesh = plsc.VectorSubcoreMesh(
        core_axis_name="c", subcore_axis_name="s", num_cores=NC, num_subcores=NS
    )

    @functools.partial(
        pl.kernel,
        out_type=jax.ShapeDtypeStruct((P, F), jnp.float32),
        mesh=mesh,
        scratch_types=[
            pltpu.VMEM((C,), jnp.int32),
            pltpu.VMEM((C, F), jnp.float32),
            pltpu.SemaphoreType.DMA,
        ],
    )
    def sc_gather(x_hbm, idx_hbm, out_hbm, idx_v, rows_v, sem):
        wid = jax.lax.axis_index("s") * NC + jax.lax.axis_index("c")
        per_w = P // NW
        base0 = wid * per_w

        def body(i, carry):
            base = base0 + i * C
            pltpu.sync_copy(idx_hbm.at[pl.ds(base, C)], idx_v)
            pltpu.async_copy(x_hbm.at[idx_v], rows_v, sem).wait()
            pltpu.sync_copy(rows_v, out_hbm.at[pl.ds(base, C)])
            return carry

        jax.lax.fori_loop(0, per_w // C, body, 0)

    @functools.partial(
        pl.kernel,
        out_type=jax.ShapeDtypeStruct((NC, NPAD, FW), jnp.float32),
        mesh=mesh,
        scratch_types=[
            pltpu.VMEM((C,), jnp.int32),
            pltpu.VMEM((C, FW), jnp.float32),
            pltpu.VMEM_SHARED((NPAD, FW), jnp.float32),
        ],
        compiler_params=pltpu.CompilerParams(use_tc_tiling_on_sc=False),
    )
    def sc_scatter(msg_hbm, dst_hbm, zeros_hbm, out_hbm, idx_v, rows_v, acc_sh):
        c = jax.lax.axis_index("c")
        s = jax.lax.axis_index("s")
        wid = s * NC + c
        rows_per_tile = NPAD // NS
        r0 = s * rows_per_tile
        pltpu.sync_copy(
            zeros_hbm.at[pl.ds(r0, rows_per_tile)],
            acc_sh.at[pl.ds(r0, rows_per_tile)],
        )
        plsc.subcore_barrier()
        per_w = P // NW
        base0 = wid * per_w

        def body(i, carry):
            base = base0 + i * C
            pltpu.sync_copy(dst_hbm.at[pl.ds(base, C)], idx_v)
            pltpu.sync_copy(msg_hbm.at[pl.ds(base, C)], rows_v)
            pltpu.sync_copy(rows_v, acc_sh.at[idx_v], add=True)
            return carry

        jax.lax.fori_loop(0, per_w // C, body, 0)
        plsc.subcore_barrier()
        pltpu.sync_copy(
            acc_sh.at[pl.ds(r0, rows_per_tile)],
            out_hbm.at[c].at[pl.ds(r0, rows_per_tile)],
        )

    return sc_gather, sc_scatter


def _tc_bmm(rel_ref, xs_ref, bd_ref, out_ref):
    m = jax.lax.dot_general(
        xs_ref[...],
        bd_ref[0],
        (((1,), (0,)), ((), ())),
        preferred_element_type=jnp.float32,
        precision=jax.lax.Precision.HIGHEST,
    )
    out_ref[:, :F] = m
    tail = jax.lax.broadcasted_iota(jnp.int32, (T, FW - F), 1)
    out_ref[:, F:] = (tail == 0).astype(jnp.float32)


def _tc_combine(a_ref, b_ref, o_ref):
    acc = a_ref[...] + b_ref[...]
    deg = jnp.maximum(acc[:, F:F + 1], 1.0)
    o_ref[...] = acc[:, :F] / deg


def kernel(x, edge_index, etype, weight):
    src = edge_index[0]
    dst = edge_index[1]

    et_s, src_s, dst_s = jax.lax.sort([etype, src, dst], num_keys=1)
    off = jnp.searchsorted(et_s, jnp.arange(R + 1, dtype=jnp.int32)).astype(
        jnp.int32
    )
    cnt = off[1:] - off[:-1]
    pcnt = ((cnt + T - 1) // T) * T
    poff = jnp.concatenate(
        [jnp.zeros((1,), jnp.int32), jnp.cumsum(pcnt).astype(jnp.int32)]
    )
    pad_before = poff[:-1] - off[:-1]
    bnd = jnp.zeros((E,), jnp.int32).at[off[:-1]].max(pad_before, mode="drop")
    pos = jnp.arange(E, dtype=jnp.int32) + jax.lax.cummax(bnd)

    i_p = jnp.arange(P, dtype=jnp.int32)
    src_p = ((i_p * 37) % N).at[pos].set(src_s, unique_indices=True)
    dst_p = (N + (i_p % (NPAD - N))).at[pos].set(dst_s, unique_indices=True)

    tstart = jnp.arange(NT, dtype=jnp.int32) * T
    tile_rel = jnp.clip(
        jnp.searchsorted(poff, tstart, side="right").astype(jnp.int32) - 1,
        0,
        R - 1,
    )

    w4 = weight.reshape(R, B, S, S)
    eye = jnp.eye(B, dtype=weight.dtype)
    bd = (w4[:, :, :, None, :] * eye[None, :, None, :, None]).reshape(R, F, F)

    sc_gather, sc_scatter = _sc_kernels()

    xs = sc_gather(x, src_p)

    grid_spec = pltpu.PrefetchScalarGridSpec(
        num_scalar_prefetch=1,
        grid=(NT,),
        in_specs=[
            pl.BlockSpec((T, F), lambda t, rel: (t, 0)),
            pl.BlockSpec((1, F, F), lambda t, rel: (rel[t], 0, 0)),
        ],
        out_specs=pl.BlockSpec((T, FW), lambda t, rel: (t, 0)),
    )
    msg = pl.pallas_call(
        _tc_bmm,
        grid_spec=grid_spec,
        out_shape=jax.ShapeDtypeStruct((P, FW), jnp.float32),
    )(tile_rel, xs, bd)

    zeros = jnp.zeros((NPAD, FW), jnp.float32)
    parts = sc_scatter(msg, dst_p, zeros)

    RT = 256
    h_pad = pl.pallas_call(
        _tc_combine,
        grid=(NPAD // RT,),
        in_specs=[
            pl.BlockSpec((RT, FW), lambda i: (i, 0)),
            pl.BlockSpec((RT, FW), lambda i: (i, 0)),
        ],
        out_specs=pl.BlockSpec((RT, F), lambda i: (i, 0)),
        out_shape=jax.ShapeDtypeStruct((NPAD, F), jnp.float32),
    )(parts[0], parts[1])
    return h_pad[:N]
